# baseline (speedup 1.0000x reference)
import functools

import jax
import jax.numpy as jnp
from jax import lax
from jax.experimental import pallas as pl
from jax.experimental.pallas import tpu as pltpu
from jax.experimental.pallas import tpu_sc as plsc

VOCAB = 100000
EMB = 128
B = 4096
L = 20
N = B * L
_K = 2
_NL = N // _K

_info = plsc.get_sparse_core_info()
_NC = _info.num_cores
_NS = _info.num_subcores
_NW = _NC * _NS
_B_PER_W = _NL // _NW
_CHUNK = 128
_N_CHUNKS = _B_PER_W // _CHUNK
_NBUF = 6

_sc_mesh = plsc.VectorSubcoreMesh(core_axis_name="c", subcore_axis_name="s")


@functools.partial(
    pl.kernel,
    mesh=_sc_mesh,
    out_type=jax.ShapeDtypeStruct((_NL, EMB), jnp.float32),
    scratch_types=[
        pltpu.VMEM((_B_PER_W,), jnp.int32),
        pltpu.VMEM((_NBUF, _CHUNK, EMB), jnp.float32),
    ]
    + [pltpu.SemaphoreType.DMA] * (2 * _NBUF),
)
def _gather_sc(idx_hbm, table_hbm, out_hbm, idx_v, rows_v, *sems):
    gsems, wsems = sems[:_NBUF], sems[_NBUF:]
    wid = lax.axis_index("s") * _NC + lax.axis_index("c")
    base = wid * _B_PER_W
    pltpu.sync_copy(idx_hbm.at[pl.ds(base, _B_PER_W)], idx_v)

    def start_g(c):
        b = c % _NBUF
        return pltpu.async_copy(
            table_hbm.at[idx_v.at[pl.ds(c * _CHUNK, _CHUNK)]], rows_v.at[b], gsems[b]
        )

    def start_w(c):
        b = c % _NBUF
        return pltpu.async_copy(
            rows_v.at[b], out_hbm.at[pl.ds(base + c * _CHUNK, _CHUNK)], wsems[b]
        )

    gcp = [start_g(c) for c in range(min(_NBUF, _N_CHUNKS))]
    wcp = [None] * _N_CHUNKS
    for c in range(_N_CHUNKS):
        gcp[c % _NBUF].wait()
        wcp[c] = start_w(c)
        nxt = c + _NBUF
        if nxt < _N_CHUNKS:
            wcp[c].wait()
            gcp[nxt % _NBUF] = start_g(nxt)
    for c in range(max(0, _N_CHUNKS - _NBUF), _N_CHUNKS):
        if wcp[c] is not None:
            wcp[c].wait()


_RB = 8192


def _mlp_body(x_ref, w1_ref, b1_ref, w2_ref, b2_ref, o_ref):
    x = x_ref[...]
    h = jnp.dot(x, w1_ref[...], preferred_element_type=jnp.float32) + b1_ref[...]
    h = 0.5 * h * (1.0 + lax.erf(h * 0.7071067811865476))
    o_ref[...] = (
        jnp.dot(h, w2_ref[...], preferred_element_type=jnp.float32) + b2_ref[...]
    )


def _mlp_part(x, W1, b1, W2, b2, prev, part):
    nb = _NL // _RB
    body = _mlp_body if prev is None else (lambda *a: _mlp_body(*a[:5], a[6]))
    specs = [
        pl.BlockSpec((_RB, EMB), lambda i: (i, 0)),
        pl.BlockSpec((EMB, EMB), lambda i: (0, 0)),
        pl.BlockSpec((1, EMB), lambda i: (0, 0)),
        pl.BlockSpec((EMB, EMB), lambda i: (0, 0)),
        pl.BlockSpec((1, EMB), lambda i: (0, 0)),
    ]
    args = (x, W1, b1, W2, b2)
    aliases = {}
    if prev is not None:
        specs.append(pl.BlockSpec(memory_space=pl.ANY))
        args = args + (prev,)
        aliases = {5: 0}
    return pl.pallas_call(
        body,
        grid=(nb,),
        in_specs=specs,
        out_specs=pl.BlockSpec((_RB, EMB), lambda i, part=part: (part * nb + i, 0)),
        out_shape=jax.ShapeDtypeStruct((N, EMB), jnp.float32),
        input_output_aliases=aliases,
    )(*args)


def kernel(te, E, W1, b1, W2, b2):
    idx = te.astype(jnp.int32).T.reshape(-1)
    b1r = b1.reshape(1, EMB)
    b2r = b2.reshape(1, EMB)
    parts = [_gather_sc(idx[k * _NL:(k + 1) * _NL], E) for k in range(_K)]
    out = None
    for k in range(_K):
        out = _mlp_part(parts[k], W1, b1r, W2, b2r, out, k)
    return out.reshape(L, B, EMB).transpose(1, 0, 2)


# speedup vs baseline: 1.0137x; 1.0137x over previous
import functools

import jax
import jax.numpy as jnp
from jax import lax
from jax.experimental import pallas as pl
from jax.experimental.pallas import tpu as pltpu
from jax.experimental.pallas import tpu_sc as plsc

VOCAB = 100000
EMB = 128
B = 4096
L = 20
N = B * L
_K = 2
_NL = N // _K

_info = plsc.get_sparse_core_info()
_NC = _info.num_cores
_NS = _info.num_subcores
_NW = _NC * _NS
_B_PER_W = _NL // _NW
_CHUNK = 128
_N_CHUNKS = _B_PER_W // _CHUNK
_NBUF = 6

_sc_mesh = plsc.VectorSubcoreMesh(core_axis_name="c", subcore_axis_name="s")


@functools.partial(
    pl.kernel,
    mesh=_sc_mesh,
    out_type=jax.ShapeDtypeStruct((_NL, EMB), jnp.float32),
    scratch_types=[
        pltpu.VMEM((_B_PER_W,), jnp.int32),
        pltpu.VMEM((_NBUF, _CHUNK, EMB), jnp.float32),
    ]
    + [pltpu.SemaphoreType.DMA] * (2 * _NBUF),
)
def _gather_sc(idx_hbm, table_hbm, out_hbm, idx_v, rows_v, *sems):
    gsems, wsems = sems[:_NBUF], sems[_NBUF:]
    wid = lax.axis_index("s") * _NC + lax.axis_index("c")
    base = wid * _B_PER_W
    pltpu.sync_copy(idx_hbm.at[pl.ds(base, _B_PER_W)], idx_v)

    def start_g(c):
        b = c % _NBUF
        return pltpu.async_copy(
            table_hbm.at[idx_v.at[pl.ds(c * _CHUNK, _CHUNK)]], rows_v.at[b], gsems[b]
        )

    def start_w(c):
        b = c % _NBUF
        return pltpu.async_copy(
            rows_v.at[b], out_hbm.at[pl.ds(base + c * _CHUNK, _CHUNK)], wsems[b]
        )

    gcp = [start_g(c) for c in range(min(_NBUF, _N_CHUNKS))]
    wcp = [None] * _N_CHUNKS
    for c in range(_N_CHUNKS):
        gcp[c % _NBUF].wait()
        wcp[c] = start_w(c)
        nxt = c + _NBUF
        if nxt < _N_CHUNKS:
            wcp[c].wait()
            gcp[nxt % _NBUF] = start_g(nxt)
    for c in range(max(0, _N_CHUNKS - _NBUF), _N_CHUNKS):
        if wcp[c] is not None:
            wcp[c].wait()


_RB = 10240


def _mlp_body(x_ref, w1_ref, b1_ref, w2_ref, b2_ref, o_ref):
    x = x_ref[...]
    h = jnp.dot(x, w1_ref[...], preferred_element_type=jnp.float32) + b1_ref[...]
    h = 0.5 * h * (1.0 + lax.erf(h * 0.7071067811865476))
    o_ref[...] = (
        jnp.dot(h, w2_ref[...], preferred_element_type=jnp.float32) + b2_ref[...]
    )


def _mlp_part(x, W1, b1, W2, b2, prev, part):
    nb = _NL // _RB
    body = _mlp_body if prev is None else (lambda *a: _mlp_body(*a[:5], a[6]))
    specs = [
        pl.BlockSpec((_RB, EMB), lambda i: (i, 0)),
        pl.BlockSpec((EMB, EMB), lambda i: (0, 0)),
        pl.BlockSpec((1, EMB), lambda i: (0, 0)),
        pl.BlockSpec((EMB, EMB), lambda i: (0, 0)),
        pl.BlockSpec((1, EMB), lambda i: (0, 0)),
    ]
    args = (x, W1, b1, W2, b2)
    aliases = {}
    if prev is not None:
        specs.append(pl.BlockSpec(memory_space=pl.ANY))
        args = args + (prev,)
        aliases = {5: 0}
    return pl.pallas_call(
        body,
        grid=(nb,),
        in_specs=specs,
        out_specs=pl.BlockSpec((_RB, EMB), lambda i, part=part: (part * nb + i, 0)),
        out_shape=jax.ShapeDtypeStruct((N, EMB), jnp.float32),
        input_output_aliases=aliases,
    )(*args)


def kernel(te, E, W1, b1, W2, b2):
    idx = te.astype(jnp.int32).T.reshape(-1)
    b1r = b1.reshape(1, EMB)
    b2r = b2.reshape(1, EMB)
    parts = [_gather_sc(idx[k * _NL:(k + 1) * _NL], E) for k in range(_K)]
    out = None
    for k in range(_K):
        out = _mlp_part(parts[k], W1, b1r, W2, b2r, out, k)
    return out.reshape(L, B, EMB).transpose(1, 0, 2)


# speedup vs baseline: 1.0309x; 1.0170x over previous
import functools

import jax
import jax.numpy as jnp
from jax import lax
from jax.experimental import pallas as pl
from jax.experimental.pallas import tpu as pltpu
from jax.experimental.pallas import tpu_sc as plsc

VOCAB = 100000
EMB = 128
B = 4096
L = 20
N = B * L
_K = 1
_NL = N // _K

_info = plsc.get_sparse_core_info()
_NC = _info.num_cores
_NS = _info.num_subcores
_NW = _NC * _NS
_B_PER_W = _NL // _NW
_CHUNK = 128
_N_CHUNKS = _B_PER_W // _CHUNK
_NBUF = 7

_sc_mesh = plsc.VectorSubcoreMesh(core_axis_name="c", subcore_axis_name="s")


@functools.partial(
    pl.kernel,
    mesh=_sc_mesh,
    out_type=jax.ShapeDtypeStruct((_NL, EMB), jnp.float32),
    scratch_types=[
        pltpu.VMEM((_B_PER_W,), jnp.int32),
        pltpu.VMEM((_NBUF, _CHUNK, EMB), jnp.float32),
    ]
    + [pltpu.SemaphoreType.DMA] * (2 * _NBUF),
)
def _gather_sc(idx_hbm, table_hbm, out_hbm, idx_v, rows_v, *sems):
    gsems, wsems = sems[:_NBUF], sems[_NBUF:]
    wid = lax.axis_index("s") * _NC + lax.axis_index("c")
    base = wid * _B_PER_W
    pltpu.sync_copy(idx_hbm.at[pl.ds(base, _B_PER_W)], idx_v)

    def start_g(c):
        b = c % _NBUF
        return pltpu.async_copy(
            table_hbm.at[idx_v.at[pl.ds(c * _CHUNK, _CHUNK)]], rows_v.at[b], gsems[b]
        )

    def start_w(c):
        b = c % _NBUF
        return pltpu.async_copy(
            rows_v.at[b], out_hbm.at[pl.ds(base + c * _CHUNK, _CHUNK)], wsems[b]
        )

    gcp = [start_g(c) for c in range(min(_NBUF, _N_CHUNKS))]
    wcp = [None] * _N_CHUNKS
    for c in range(_N_CHUNKS):
        gcp[c % _NBUF].wait()
        wcp[c] = start_w(c)
        nxt = c + _NBUF
        if nxt < _N_CHUNKS:
            wcp[c].wait()
            gcp[nxt % _NBUF] = start_g(nxt)
    for c in range(max(0, _N_CHUNKS - _NBUF), _N_CHUNKS):
        if wcp[c] is not None:
            wcp[c].wait()


_RB = 20480


def _mlp_body(x_ref, w1_ref, b1_ref, w2_ref, b2_ref, o_ref):
    x = x_ref[...]
    h = jnp.dot(x, w1_ref[...], preferred_element_type=jnp.float32) + b1_ref[...]
    h = 0.5 * h * (1.0 + lax.erf(h * 0.7071067811865476))
    o_ref[...] = (
        jnp.dot(h, w2_ref[...], preferred_element_type=jnp.float32) + b2_ref[...]
    )


def _mlp_part(x, W1, b1, W2, b2, prev, part):
    nb = _NL // _RB
    body = _mlp_body if prev is None else (lambda *a: _mlp_body(*a[:5], a[6]))
    specs = [
        pl.BlockSpec((_RB, EMB), lambda i: (i, 0)),
        pl.BlockSpec((EMB, EMB), lambda i: (0, 0)),
        pl.BlockSpec((1, EMB), lambda i: (0, 0)),
        pl.BlockSpec((EMB, EMB), lambda i: (0, 0)),
        pl.BlockSpec((1, EMB), lambda i: (0, 0)),
    ]
    args = (x, W1, b1, W2, b2)
    aliases = {}
    if prev is not None:
        specs.append(pl.BlockSpec(memory_space=pl.ANY))
        args = args + (prev,)
        aliases = {5: 0}
    return pl.pallas_call(
        body,
        grid=(nb,),
        in_specs=specs,
        out_specs=pl.BlockSpec((_RB, EMB), lambda i, part=part: (part * nb + i, 0)),
        out_shape=jax.ShapeDtypeStruct((N, EMB), jnp.float32),
        input_output_aliases=aliases,
    )(*args)


def kernel(te, E, W1, b1, W2, b2):
    idx = te.astype(jnp.int32).T.reshape(-1)
    b1r = b1.reshape(1, EMB)
    b2r = b2.reshape(1, EMB)
    parts = [_gather_sc(idx[k * _NL:(k + 1) * _NL], E) for k in range(_K)]
    out = None
    for k in range(_K):
        out = _mlp_part(parts[k], W1, b1r, W2, b2r, out, k)
    return out.reshape(L, B, EMB).transpose(1, 0, 2)
